# baseline (speedup 1.0000x reference)
import functools

import jax
import jax.numpy as jnp
import numpy as np
from jax import lax
from jax.experimental import pallas as pl
from jax.experimental.pallas import tpu as pltpu
from jax.experimental.pallas import tpu_sc as plsc

_FIELD_DIMS = [100000] * 26
_F = len(_FIELD_DIMS)
_E = 16
_B = 16384
_N = _B * _F
_OFFSETS_NP = np.concatenate(([0], np.cumsum(_FIELD_DIMS[:-1]))).astype(np.int32)

_NC, _NS, _L = 2, 16, 16
_NW = _NC * _NS
_PER_W = _N // _NW
_CHUNK = 3328
_NCHUNK = _PER_W // _CHUNK
_OFFS_TILED_NP = np.tile(_OFFSETS_NP, _CHUNK // _F)


def _make_gather():
    mesh = plsc.VectorSubcoreMesh(core_axis_name="c", subcore_axis_name="s")

    @functools.partial(
        pl.kernel,
        mesh=mesh,
        out_type=jax.ShapeDtypeStruct((_N, _E), jnp.float32),
        compiler_params=pltpu.CompilerParams(use_tc_tiling_on_sc=False),
        scratch_types=[
            pltpu.VMEM((_CHUNK,), jnp.int32),
            pltpu.VMEM((_CHUNK,), jnp.int32),
            pltpu.VMEM((_CHUNK,), jnp.int32),
            pltpu.VMEM((_CHUNK, _E), jnp.float32),
            pltpu.SemaphoreType.DMA,
        ],
    )
    def gather_kernel(x_hbm, offs_hbm, table_hbm, out_hbm,
                      off_v, x_v, idx_v, rows_v, sem):
        wid = lax.axis_index("s") * _NC + lax.axis_index("c")
        base = wid * _PER_W
        pltpu.sync_copy(offs_hbm, off_v)
        for c in range(_NCHUNK):
            start = base + c * _CHUNK
            pltpu.sync_copy(x_hbm.at[pl.ds(start, _CHUNK)], x_v)

            def add_body(j, carry):
                sl = pl.ds(j * _L, _L)
                idx_v[sl] = x_v[sl] + off_v[sl]
                return carry

            lax.fori_loop(0, _CHUNK // _L, add_body, 0)
            pltpu.async_copy(table_hbm.at[idx_v], rows_v, sem).wait()
            pltpu.sync_copy(rows_v, out_hbm.at[pl.ds(start, _CHUNK)])

    return gather_kernel


_GATHER = _make_gather()


def kernel(x, table):
    x_flat = x.reshape(_N)
    offs = jnp.asarray(_OFFS_TILED_NP)
    out = _GATHER(x_flat, offs, table)
    return out.reshape(_B, _F, _E)


# speedup vs baseline: 1.0017x; 1.0017x over previous
import functools

import jax
import jax.numpy as jnp
import numpy as np
from jax import lax
from jax.experimental import pallas as pl
from jax.experimental.pallas import tpu as pltpu
from jax.experimental.pallas import tpu_sc as plsc

_FIELD_DIMS = [100000] * 26
_F = len(_FIELD_DIMS)
_E = 16
_B = 16384
_N = _B * _F
_OFFSETS_NP = np.concatenate(([0], np.cumsum(_FIELD_DIMS[:-1]))).astype(np.int32)

_NC, _NS, _L = 2, 16, 16
_NW = _NC * _NS
_PER_W = _N // _NW
_CHUNK = 3328
_NCHUNK = _PER_W // _CHUNK
_OFFS_TILED_NP = np.tile(_OFFSETS_NP, _CHUNK // _F)


def _make_gather():
    mesh = plsc.VectorSubcoreMesh(core_axis_name="c", subcore_axis_name="s")

    @functools.partial(
        pl.kernel,
        mesh=mesh,
        out_type=jax.ShapeDtypeStruct((_N, _E), jnp.float32),
        compiler_params=pltpu.CompilerParams(use_tc_tiling_on_sc=False),
        scratch_types=[
            pltpu.VMEM((_CHUNK,), jnp.int32),
            pltpu.VMEM((_CHUNK,), jnp.int32),
            pltpu.VMEM((_CHUNK,), jnp.int32),
            pltpu.VMEM((_CHUNK, _E), jnp.float32),
            pltpu.VMEM((_CHUNK, _E), jnp.float32),
            pltpu.SemaphoreType.DMA,
            pltpu.SemaphoreType.DMA,
            pltpu.SemaphoreType.DMA,
            pltpu.SemaphoreType.DMA,
        ],
    )
    def gather_kernel(x_hbm, offs_hbm, table_hbm, out_hbm,
                      off_v, idx_a, idx_b, rows_a, rows_b,
                      gsem_a, gsem_b, ssem_a, ssem_b):
        wid = lax.axis_index("s") * _NC + lax.axis_index("c")
        base = wid * _PER_W
        pltpu.sync_copy(offs_hbm, off_v)
        idx = [idx_a, idx_b]
        rows = [rows_a, rows_b]
        gsem = [gsem_a, gsem_b]
        ssem = [ssem_a, ssem_b]
        gh = [None, None]
        sh = [None, None]
        for c in range(_NCHUNK):
            b = c % 2
            if c >= 2:
                sh[b].wait()
            start = base + c * _CHUNK
            pltpu.sync_copy(x_hbm.at[pl.ds(start, _CHUNK)], idx[b])

            def add_body(j, carry, _ib=idx[b]):
                sl = pl.ds(j * _L, _L)
                _ib[sl] = _ib[sl] + off_v[sl]
                return carry

            lax.fori_loop(0, _CHUNK // _L, add_body, 0)
            gh[b] = pltpu.async_copy(table_hbm.at[idx[b]], rows[b], gsem[b])
            if c >= 1:
                pb = 1 - b
                gh[pb].wait()
                sh[pb] = pltpu.async_copy(
                    rows[pb],
                    out_hbm.at[pl.ds(base + (c - 1) * _CHUNK, _CHUNK)],
                    ssem[pb],
                )
        last = (_NCHUNK - 1) % 2
        gh[last].wait()
        sh[last] = pltpu.async_copy(
            rows[last],
            out_hbm.at[pl.ds(base + (_NCHUNK - 1) * _CHUNK, _CHUNK)],
            ssem[last],
        )
        sh[0].wait()
        sh[1].wait()

    return gather_kernel


_GATHER = _make_gather()


def kernel(x, table):
    x_flat = x.reshape(_N)
    offs = jnp.asarray(_OFFS_TILED_NP)
    out = _GATHER(x_flat, offs, table)
    return out.reshape(_B, _F, _E)
